# Initial kernel scaffold; baseline (speedup 1.0000x reference)
import jax
import jax.numpy as jnp
from jax.experimental import pallas as pl


def kernel(s_feats, q_points, s_points, neighbor_indices):
    raise NotImplementedError("write your pallas kernel here")


# speedup vs baseline: 11.1174x; 11.1174x over previous
import functools

import jax
import jax.numpy as jnp
from jax import lax
from jax.experimental import pallas as pl
from jax.experimental.pallas import tpu as pltpu
from jax.experimental.pallas import tpu_sc as plsc

KNN = 3
EPS = 1e-8
L = 16
NC = 2
NS = 16
NW = NC * NS


@functools.partial(jax.jit, static_argnums=(0, 1, 2))
def _sc_call(S, Q, C, s_feats, qx, qy, qz, spx, spy, spz, i0, i1, i2):
    QPW = Q // NW
    FB = 32
    NFB = QPW // FB
    CBN = C // L

    mesh = plsc.VectorSubcoreMesh(core_axis_name="c", subcore_axis_name="s")

    @functools.partial(
        pl.kernel,
        out_type=jax.ShapeDtypeStruct((Q, C), jnp.float32),
        mesh=mesh,
        compiler_params=pltpu.CompilerParams(needs_layout_passes=False),
        scratch_types=[
            pltpu.VMEM((QPW,), jnp.int32),
            pltpu.VMEM((QPW,), jnp.int32),
            pltpu.VMEM((QPW,), jnp.int32),
            pltpu.VMEM((KNN, QPW), jnp.float32),
            pltpu.VMEM((QPW,), jnp.float32),
            pltpu.VMEM((QPW,), jnp.float32),
            pltpu.VMEM((QPW,), jnp.float32),
            pltpu.VMEM((S,), jnp.float32),
            pltpu.VMEM((S,), jnp.float32),
            pltpu.VMEM((S,), jnp.float32),
            pltpu.VMEM((KNN, FB, C), jnp.float32),
            pltpu.VMEM((FB, C), jnp.float32),
            pltpu.SemaphoreType.DMA,
            pltpu.SemaphoreType.DMA,
            pltpu.SemaphoreType.DMA,
        ],
    )
    def knn_kernel(feats_hbm, qx_hbm, qy_hbm, qz_hbm, spx_hbm, spy_hbm,
                   spz_hbm, i0_hbm, i1_hbm, i2_hbm, out_hbm,
                   idx0_v, idx1_v, idx2_v, w_v, qx_v, qy_v, qz_v,
                   spx_v, spy_v, spz_v, r_v, out_v, sem0, sem1, sem2):
        wid = lax.axis_index("s") * NC + lax.axis_index("c")
        base = wid * QPW
        idx_refs = (idx0_v, idx1_v, idx2_v)
        sems = (sem0, sem1, sem2)

        pltpu.sync_copy(spx_hbm, spx_v)
        pltpu.sync_copy(spy_hbm, spy_v)
        pltpu.sync_copy(spz_hbm, spz_v)
        for k, src in enumerate((i0_hbm, i1_hbm, i2_hbm)):
            pltpu.sync_copy(src.at[pl.ds(base, QPW)], idx_refs[k])
        pltpu.sync_copy(qx_hbm.at[pl.ds(base, QPW)], qx_v)
        pltpu.sync_copy(qy_hbm.at[pl.ds(base, QPW)], qy_v)
        pltpu.sync_copy(qz_hbm.at[pl.ds(base, QPW)], qz_v)

        zero_i = jnp.zeros((L,), jnp.int32)

        @pl.loop(0, QPW // L)
        def _p1(g):
            sl = pl.ds(g * L, L)
            qxv = qx_v[sl]
            qyv = qy_v[sl]
            qzv = qz_v[sl]
            ws = []
            for k in range(KNN):
                iv = idx_refs[k][sl]
                sx = plsc.load_gather(spx_v, [iv])
                sy = plsc.load_gather(spy_v, [iv])
                sz = plsc.load_gather(spz_v, [iv])
                dx = qxv - sx
                dy = qyv - sy
                dz = qzv - sz
                d2 = dx * dx + dy * dy + dz * dz
                ws.append(1.0 / (d2 + EPS))
            wsum = ws[0] + ws[1] + ws[2]
            for k in range(KNN):
                w_v[k, sl] = ws[k] / wsum

        @pl.loop(0, NFB)
        def _p2(fb):
            qb = fb * FB
            cps = [
                pltpu.async_copy(
                    feats_hbm.at[idx_refs[k].at[pl.ds(qb, FB)]], r_v.at[k], sems[k])
                for k in range(KNN)
            ]
            for cp in cps:
                cp.wait()

            @pl.loop(0, FB)
            def _q(qi):
                widx = jnp.full((L,), qb + qi, dtype=jnp.int32)
                w0 = plsc.load_gather(w_v, [zero_i, widx])
                w1 = plsc.load_gather(w_v, [zero_i + 1, widx])
                w2 = plsc.load_gather(w_v, [zero_i + 2, widx])
                for cb in range(CBN):
                    cs = pl.ds(cb * L, L)
                    out_v[qi, cs] = (w0 * r_v[0, qi, cs]
                                     + w1 * r_v[1, qi, cs]
                                     + w2 * r_v[2, qi, cs])

            pltpu.sync_copy(out_v, out_hbm.at[pl.ds(base + qb, FB)])

    return knn_kernel(s_feats, qx, qy, qz, spx, spy, spz, i0, i1, i2)


def kernel(s_feats, q_points, s_points, neighbor_indices):
    S, C = s_feats.shape
    Q = q_points.shape[0]
    qp = q_points.astype(jnp.float32)
    sp = s_points.astype(jnp.float32)
    ni = neighbor_indices.astype(jnp.int32)
    return _sc_call(S, Q, C, s_feats.astype(jnp.float32),
                    qp[:, 0], qp[:, 1], qp[:, 2],
                    sp[:, 0], sp[:, 1], sp[:, 2],
                    ni[:, 0], ni[:, 1], ni[:, 2])


# speedup vs baseline: 18.2298x; 1.6397x over previous
import functools

import jax
import jax.numpy as jnp
from jax import lax
from jax.experimental import pallas as pl
from jax.experimental.pallas import tpu as pltpu
from jax.experimental.pallas import tpu_sc as plsc

KNN = 3
EPS = 1e-8
L = 16
NC = 2
NS = 16
NW = NC * NS


@functools.partial(jax.jit, static_argnums=(0, 1, 2))
def _sc_call(S, Q, C, s_feats, qx, qy, qz, spx, spy, spz, i0, i1, i2):
    QPW = Q // NW
    FB = 32
    NFB = QPW // FB
    CBN = C // L

    mesh = plsc.VectorSubcoreMesh(core_axis_name="c", subcore_axis_name="s")

    @functools.partial(
        pl.kernel,
        out_type=jax.ShapeDtypeStruct((Q, C), jnp.float32),
        mesh=mesh,
        compiler_params=pltpu.CompilerParams(needs_layout_passes=False),
        scratch_types=[
            pltpu.VMEM((QPW,), jnp.int32),
            pltpu.VMEM((QPW,), jnp.int32),
            pltpu.VMEM((QPW,), jnp.int32),
            pltpu.VMEM((KNN, QPW), jnp.float32),
            pltpu.SemaphoreType.DMA,
            pltpu.SemaphoreType.DMA,
            pltpu.SemaphoreType.DMA,
            pltpu.SemaphoreType.DMA,
        ],
    )
    def knn_kernel(feats_hbm, qx_hbm, qy_hbm, qz_hbm, spx_hbm, spy_hbm,
                   spz_hbm, i0_hbm, i1_hbm, i2_hbm, out_hbm,
                   idx0_v, idx1_v, idx2_v, w_v, sgA, sgB, soA, soB):
        wid = lax.axis_index("s") * NC + lax.axis_index("c")
        base = wid * QPW
        idx_refs = (idx0_v, idx1_v, idx2_v)

        for k, src in enumerate((i0_hbm, i1_hbm, i2_hbm)):
            pltpu.sync_copy(src.at[pl.ds(base, QPW)], idx_refs[k])

        zero_i = jnp.zeros((L,), jnp.int32)

        def _phase1(qx_v, qy_v, qz_v, spx_v, spy_v, spz_v):
            pltpu.sync_copy(qx_hbm.at[pl.ds(base, QPW)], qx_v)
            pltpu.sync_copy(qy_hbm.at[pl.ds(base, QPW)], qy_v)
            pltpu.sync_copy(qz_hbm.at[pl.ds(base, QPW)], qz_v)
            pltpu.sync_copy(spx_hbm, spx_v)
            pltpu.sync_copy(spy_hbm, spy_v)
            pltpu.sync_copy(spz_hbm, spz_v)

            @pl.loop(0, QPW // L)
            def _p1(g):
                sl = pl.ds(g * L, L)
                qxv = qx_v[sl]
                qyv = qy_v[sl]
                qzv = qz_v[sl]
                ws = []
                for k in range(KNN):
                    iv = idx_refs[k][sl]
                    sx = plsc.load_gather(spx_v, [iv])
                    sy = plsc.load_gather(spy_v, [iv])
                    sz = plsc.load_gather(spz_v, [iv])
                    dx = qxv - sx
                    dy = qyv - sy
                    dz = qzv - sz
                    d2 = dx * dx + dy * dy + dz * dz
                    ws.append(1.0 / (d2 + EPS))
                wsum = ws[0] + ws[1] + ws[2]
                for k in range(KNN):
                    w_v[k, sl] = ws[k] / wsum

        pl.run_scoped(
            _phase1,
            pltpu.VMEM((QPW,), jnp.float32),
            pltpu.VMEM((QPW,), jnp.float32),
            pltpu.VMEM((QPW,), jnp.float32),
            pltpu.VMEM((S,), jnp.float32),
            pltpu.VMEM((S,), jnp.float32),
            pltpu.VMEM((S,), jnp.float32),
        )

        def _issue(qb, r, sg):
            for k in range(KNN):
                pltpu.async_copy(
                    feats_hbm.at[idx_refs[k].at[pl.ds(qb, FB)]], r.at[k], sg)

        def _wait_g(qb, r, sg):
            for k in range(KNN):
                pltpu.make_async_copy(
                    feats_hbm.at[idx_refs[k].at[pl.ds(qb, FB)]], r.at[k],
                    sg).wait()

        def _wait_o(o, so):
            pltpu.make_async_copy(o, out_hbm.at[pl.ds(base, FB)], so).wait()

        def _compute(qb, r, o):
            @pl.loop(0, FB)
            def _q(qi):
                widx = jnp.full((L,), qb + qi, dtype=jnp.int32)
                w0 = plsc.load_gather(w_v, [zero_i, widx])
                w1 = plsc.load_gather(w_v, [zero_i + 1, widx])
                w2 = plsc.load_gather(w_v, [zero_i + 2, widx])
                for cb in range(CBN):
                    cs = pl.ds(cb * L, L)
                    o[qi, cs] = (w0 * r[0, qi, cs]
                                 + w1 * r[1, qi, cs]
                                 + w2 * r[2, qi, cs])

        def _phase2(rA, rB, outA, outB):
            slots = ((rA, outA, sgA, soA), (rB, outB, sgB, soB))
            _issue(0, rA, sgA)
            _issue(FB, rB, sgB)

            @pl.loop(0, NFB // 2)
            def _p2(p):
                for off, (r, o, sg, so) in enumerate(slots):
                    qb = (2 * p + off) * FB
                    _wait_g(qb, r, sg)

                    @pl.when(p > 0)
                    def _():
                        _wait_o(o, so)

                    _compute(qb, r, o)
                    pltpu.async_copy(o, out_hbm.at[pl.ds(base + qb, FB)], so)
                    nqb = qb + 2 * FB

                    @pl.when(nqb < QPW)
                    def _():
                        _issue(nqb, r, sg)

            _wait_o(outA, soA)
            _wait_o(outB, soB)

        pl.run_scoped(
            _phase2,
            pltpu.VMEM((KNN, FB, C), jnp.float32),
            pltpu.VMEM((KNN, FB, C), jnp.float32),
            pltpu.VMEM((FB, C), jnp.float32),
            pltpu.VMEM((FB, C), jnp.float32),
        )

    return knn_kernel(s_feats, qx, qy, qz, spx, spy, spz, i0, i1, i2)


def kernel(s_feats, q_points, s_points, neighbor_indices):
    S, C = s_feats.shape
    Q = q_points.shape[0]
    qp = q_points.astype(jnp.float32)
    sp = s_points.astype(jnp.float32)
    ni = neighbor_indices.astype(jnp.int32)
    return _sc_call(S, Q, C, s_feats.astype(jnp.float32),
                    qp[:, 0], qp[:, 1], qp[:, 2],
                    sp[:, 0], sp[:, 1], sp[:, 2],
                    ni[:, 0], ni[:, 1], ni[:, 2])
